# Initial kernel scaffold; baseline (speedup 1.0000x reference)
import jax
import jax.numpy as jnp
from jax.experimental import pallas as pl


def kernel(node_embeddings, edge_index, node_depths, W0, b0, W1, b1, W2, b2, Wg1, bg1, Wg2, bg2, Wa, ba, Wf, bf):
    raise NotImplementedError("write your pallas kernel here")


# speedup vs baseline: 2.6482x; 2.6482x over previous
import functools

import jax
import jax.numpy as jnp
from jax import lax
from jax.experimental import pallas as pl
from jax.experimental.pallas import tpu as pltpu
from jax.experimental.pallas import tpu_sc as plsc

N = 10000
D = 128
E = 320000
MAX_HOP = 3

NC = 2
NS = 16
NW = NC * NS

NP = 10240
RPS = NP // NS
EW = 10240
CB = 128
CH = EW // CB
DP = NP // NW

_mesh = plsc.VectorSubcoreMesh(
    core_axis_name="c", subcore_axis_name="s", num_cores=NC, num_subcores=NS)


def _scatter_body(with_extras, *refs):
    if with_extras:
        (feat, row_h, col_h, dep_h, part_h, cnt_h, dmax_h,
         row_v, col_v, rows_v, zz_v, z1_v, ones_v, dm_v, dep_v,
         acc_s, cnt_s, sem) = refs
    else:
        (feat, row_h, col_h, part_h,
         row_v, col_v, rows_v, zz_v, acc_s, sem) = refs

    cid = lax.axis_index("c")
    sid = lax.axis_index("s")
    wid = sid * NC + cid
    base = sid * RPS

    zero16f = jnp.zeros((16,), jnp.float32)

    def _zbody(r, carry):
        for j in range(D // 16):
            zz_v[r, pl.ds(j * 16, 16)] = zero16f
        return carry
    lax.fori_loop(0, 64, _zbody, 0)
    for k in range(RPS // 64):
        pltpu.sync_copy(zz_v, acc_s.at[pl.ds(base + k * 64, 64)])

    if with_extras:
        def _z1body(r, carry):
            z1_v[pl.ds(r * 16, 16)] = zero16f
            return carry
        lax.fori_loop(0, RPS // 16, _z1body, 0)
        pltpu.sync_copy(z1_v, cnt_s.at[pl.ds(base, RPS)])

        one16f = jnp.ones((16,), jnp.float32)
        for j in range(CB // 16):
            ones_v[pl.ds(j * 16, 16)] = one16f

        pltpu.sync_copy(dep_h.at[wid], dep_v)
        m = dep_v[pl.ds(0, 16)]
        for t in range(1, DP // 16):
            m = jnp.maximum(m, dep_v[pl.ds(t * 16, 16)])
        dm_v[pl.ds(0, 16)] = m
        pltpu.sync_copy(dm_v, dmax_h.at[wid])

    pltpu.sync_copy(row_h.at[wid], row_v)
    pltpu.sync_copy(col_h.at[wid], col_v)

    plsc.subcore_barrier()

    def _ebody(j, carry):
        pltpu.async_copy(feat.at[row_v.at[j]], rows_v, sem).wait()
        pltpu.sync_copy(rows_v, acc_s.at[col_v.at[j]], add=True)
        if with_extras:
            pltpu.sync_copy(ones_v, cnt_s.at[col_v.at[j]], add=True)
        return carry
    lax.fori_loop(0, CH, _ebody, 0)

    plsc.subcore_barrier()
    pltpu.sync_copy(acc_s.at[pl.ds(base, RPS)],
                    part_h.at[cid, pl.ds(base, RPS)])
    if with_extras:
        pltpu.sync_copy(cnt_s.at[pl.ds(base, RPS)],
                        cnt_h.at[cid, pl.ds(base, RPS)])


def _build_scatter(with_extras):
    if with_extras:
        out_type = (jax.ShapeDtypeStruct((NC, NP, D), jnp.float32),
                    jax.ShapeDtypeStruct((NC, NP), jnp.float32),
                    jax.ShapeDtypeStruct((NW, 16), jnp.int32))
        scratch = [pltpu.VMEM((CH, CB), jnp.int32),
                   pltpu.VMEM((CH, CB), jnp.int32),
                   pltpu.VMEM((CB, D), jnp.float32),
                   pltpu.VMEM((64, D), jnp.float32),
                   pltpu.VMEM((RPS,), jnp.float32),
                   pltpu.VMEM((CB,), jnp.float32),
                   pltpu.VMEM((16,), jnp.int32),
                   pltpu.VMEM((DP,), jnp.int32),
                   pltpu.VMEM_SHARED((NP, D), jnp.float32),
                   pltpu.VMEM_SHARED((NP,), jnp.float32),
                   pltpu.SemaphoreType.DMA]
    else:
        out_type = jax.ShapeDtypeStruct((NC, NP, D), jnp.float32)
        scratch = [pltpu.VMEM((CH, CB), jnp.int32),
                   pltpu.VMEM((CH, CB), jnp.int32),
                   pltpu.VMEM((CB, D), jnp.float32),
                   pltpu.VMEM((64, D), jnp.float32),
                   pltpu.VMEM_SHARED((NP, D), jnp.float32),
                   pltpu.SemaphoreType.DMA]
    return pl.kernel(functools.partial(_scatter_body, with_extras),
                     out_type=out_type, mesh=_mesh, scratch_types=scratch)


_scatter_extras = _build_scatter(True)
_scatter_plain = _build_scatter(False)


BC = 1024


def _combine_body(p_ref, c0_ref, c1_ref, o_ref):
    r = 1.0 / jnp.maximum(c0_ref[...] + c1_ref[...], 1.0)
    o_ref[...] = (p_ref[0] + p_ref[1]) * r


def _combine(part, c0, c1):
    return pl.pallas_call(
        _combine_body,
        grid=(NP // BC,),
        in_specs=[pl.BlockSpec((NC, BC, D), lambda i: (0, i, 0)),
                  pl.BlockSpec((BC, 1), lambda i: (i, 0)),
                  pl.BlockSpec((BC, 1), lambda i: (i, 0))],
        out_specs=pl.BlockSpec((BC, D), lambda i: (i, 0)),
        out_shape=jax.ShapeDtypeStruct((NP, D), jnp.float32),
    )(part, c0, c1)


BF = 2000


def _fusion_body(x_ref, a1_ref, p2_ref, c0_ref, c1_ref,
                 w0_ref, b0_ref, w1_ref, b1_ref, w2_ref, b2_ref,
                 wa_ref, ba_ref, wf0_ref, wf1_ref, wf2_ref, bf_ref, o_ref):
    f32 = jnp.float32
    r = 1.0 / jnp.maximum(c0_ref[...] + c1_ref[...], 1.0)
    a2 = (p2_ref[0] + p2_ref[1]) * r
    h0 = jnp.dot(x_ref[...], w0_ref[...], preferred_element_type=f32) + b0_ref[...]
    h1 = jnp.dot(a1_ref[...], w1_ref[...], preferred_element_type=f32) + b1_ref[...]
    h2 = jnp.dot(a2, w2_ref[...], preferred_element_type=f32) + b2_ref[...]
    wa = wa_ref[...]
    logits = (jnp.dot(h0, wa[0:D], preferred_element_type=f32)
              + jnp.dot(h1, wa[D:2 * D], preferred_element_type=f32)
              + jnp.dot(h2, wa[2 * D:3 * D], preferred_element_type=f32)
              + ba_ref[...])
    m = jnp.max(logits, axis=-1, keepdims=True)
    e = jnp.exp(logits - m)
    att = e / jnp.sum(e, axis=-1, keepdims=True)
    o_ref[...] = (jnp.dot(h0 * att[:, 0:1], wf0_ref[...], preferred_element_type=f32)
                  + jnp.dot(h1 * att[:, 1:2], wf1_ref[...], preferred_element_type=f32)
                  + jnp.dot(h2 * att[:, 2:3], wf2_ref[...], preferred_element_type=f32)
                  + bf_ref[...])


def _fusion(x, a1, p2, c0, c1, w0, b0, w1, b1, w2, b2, wa, ba, wf0, wf1, wf2, bf):
    full = lambda shape: pl.BlockSpec(shape, lambda i: tuple(0 for _ in shape))
    return pl.pallas_call(
        _fusion_body,
        grid=(N // BF,),
        in_specs=[pl.BlockSpec((BF, D), lambda i: (i, 0)),
                  pl.BlockSpec((BF, D), lambda i: (i, 0)),
                  pl.BlockSpec((NC, BF, D), lambda i: (0, i, 0)),
                  pl.BlockSpec((BF, 1), lambda i: (i, 0)),
                  pl.BlockSpec((BF, 1), lambda i: (i, 0)),
                  full((D, D)), full((1, D)),
                  full((D, D)), full((1, D)),
                  full((D, D)), full((1, D)),
                  full((3 * D, D)), full((1, D)),
                  full((D, D)), full((D, D)), full((D, D)), full((1, D))],
        out_specs=pl.BlockSpec((BF, D), lambda i: (i, 0)),
        out_shape=jax.ShapeDtypeStruct((N, D), jnp.float32),
    )(x, a1, p2, c0, c1, w0, b0, w1, b1, w2, b2, wa, ba, wf0, wf1, wf2, bf)


def kernel(node_embeddings, edge_index, node_depths, W0, b0, W1, b1, W2, b2,
           Wg1, bg1, Wg2, bg2, Wa, ba, Wf, bf):
    f32 = jnp.float32
    i32 = jnp.int32

    Xp = jnp.concatenate([node_embeddings, jnp.zeros((NP - N, D), f32)], axis=0)
    pad_e = NW * EW - E
    rowp = jnp.concatenate([edge_index[0], jnp.full((pad_e,), N, i32)])
    colp = jnp.concatenate([edge_index[1], jnp.full((pad_e,), NP - 1, i32)])
    row_r = rowp.reshape(NW, CH, CB)
    col_r = colp.reshape(NW, CH, CB)
    dep_r = jnp.concatenate(
        [node_depths, jnp.zeros((NP - N,), i32)]).reshape(NW, DP)

    part1, cnt, dmax = _scatter_extras(Xp, row_r, col_r, dep_r)

    max_depth = jnp.max(dmax).astype(f32)
    sf = jnp.stack([jnp.asarray(N / 5000.0, f32), max_depth / 20.0])[None, :]
    g = jax.nn.relu(sf @ Wg1.T + bg1)
    sw = jax.nn.sigmoid(g @ Wg2.T + bg2)[0]

    c0 = cnt[0][:, None]
    c1 = cnt[1][:, None]
    agg1 = _combine(part1, c0, c1)

    part2 = _scatter_plain(agg1, row_r, col_r)

    w0t = W0.T * sw[0]
    w1t = W1.T * sw[1]
    w2t = W2.T * sw[2]
    b0s = (b0 * sw[0])[None, :]
    b1s = (b1 * sw[1])[None, :]
    b2s = (b2 * sw[2])[None, :]
    waP = jnp.zeros((3 * D, D), f32).at[:, :MAX_HOP].set(Wa.T)
    baP = jnp.full((1, D), -1e30, f32).at[0, :MAX_HOP].set(ba)
    wfT = Wf.T
    return _fusion(node_embeddings, agg1, part2, c0, c1,
                   w0t, b0s, w1t, b1s, w2t, b2s, waP, baP,
                   wfT[0:D], wfT[D:2 * D], wfT[2 * D:3 * D], bf[None, :])


# speedup vs baseline: 2.9474x; 1.1130x over previous
import functools

import jax
import jax.numpy as jnp
from jax import lax
from jax.experimental import pallas as pl
from jax.experimental.pallas import tpu as pltpu
from jax.experimental.pallas import tpu_sc as plsc

N = 10000
D = 128
E = 320000
MAX_HOP = 3

NC = 2
NS = 16
NW = NC * NS

NP = 10240
RPS = NP // NS
EW = 10240
CB = 128
CH = EW // CB
G = 40
DP = NP // NW

_mesh = plsc.VectorSubcoreMesh(
    core_axis_name="c", subcore_axis_name="s", num_cores=NC, num_subcores=NS)


def _scatter_body(with_extras, *refs):
    if with_extras:
        (feat, row_h, col_h, dep_h, part_h, cnt_h, dmax_h,
         row_v, col_v, rows_a, rows_b, z1_v, ones_v, dm_v, dep_v,
         acc_s, cnt_s, gsem_a, gsem_b) = refs
    else:
        (feat, row_h, col_h, part_h,
         row_v, col_v, rows_a, rows_b, acc_s, gsem_a, gsem_b) = refs

    cid = lax.axis_index("c")
    sid = lax.axis_index("s")
    wid = sid * NC + cid
    base = sid * RPS

    zero16f = jnp.zeros((16,), jnp.float32)

    def _zbody(r, carry):
        for j in range(D // 16):
            rows_a[r, pl.ds(j * 16, 16)] = zero16f
        return carry
    lax.fori_loop(0, CB, _zbody, 0)
    for k in range(RPS // CB):
        pltpu.sync_copy(rows_a, acc_s.at[pl.ds(base + k * CB, CB)])

    if with_extras:
        def _z1body(r, carry):
            z1_v[pl.ds(r * 16, 16)] = zero16f
            return carry
        lax.fori_loop(0, RPS // 16, _z1body, 0)
        pltpu.sync_copy(z1_v, cnt_s.at[pl.ds(base, RPS)])

        one16f = jnp.ones((16,), jnp.float32)
        for j in range(CB // 16):
            ones_v[pl.ds(j * 16, 16)] = one16f

        pltpu.sync_copy(dep_h.at[wid], dep_v)
        m = dep_v[pl.ds(0, 16)]
        for t in range(1, DP // 16):
            m = jnp.maximum(m, dep_v[pl.ds(t * 16, 16)])
        dm_v[pl.ds(0, 16)] = m
        pltpu.sync_copy(dm_v, dmax_h.at[wid])

    plsc.subcore_barrier()

    for sci in range(CH // G):
        pltpu.sync_copy(row_h.at[wid, pl.ds(sci * G, G)], row_v)
        pltpu.sync_copy(col_h.at[wid, pl.ds(sci * G, G)], col_v)
        pltpu.async_copy(feat.at[row_v.at[0]], rows_a, gsem_a)
        pltpu.async_copy(feat.at[row_v.at[1]], rows_b, gsem_b)

        def _step(j, rows, gsem):
            pltpu.make_async_copy(feat.at[row_v.at[j]], rows, gsem).wait()
            pltpu.sync_copy(rows, acc_s.at[col_v.at[j]], add=True)
            if with_extras:
                pltpu.sync_copy(ones_v, cnt_s.at[col_v.at[j]], add=True)

            @pl.when(j < G - 2)
            def _():
                pltpu.async_copy(feat.at[row_v.at[j + 2]], rows, gsem)

        def _pair(g, carry):
            _step(2 * g, rows_a, gsem_a)
            _step(2 * g + 1, rows_b, gsem_b)
            return carry
        lax.fori_loop(0, G // 2, _pair, 0)

    plsc.subcore_barrier()
    pltpu.sync_copy(acc_s.at[pl.ds(base, RPS)],
                    part_h.at[cid, pl.ds(base, RPS)])
    if with_extras:
        pltpu.sync_copy(cnt_s.at[pl.ds(base, RPS)],
                        cnt_h.at[cid, pl.ds(base, RPS)])


def _build_scatter(with_extras):
    if with_extras:
        out_type = (jax.ShapeDtypeStruct((NC, NP, D), jnp.float32),
                    jax.ShapeDtypeStruct((NC, NP), jnp.float32),
                    jax.ShapeDtypeStruct((NW, 16), jnp.int32))
        scratch = [pltpu.VMEM((G, CB), jnp.int32),
                   pltpu.VMEM((G, CB), jnp.int32),
                   pltpu.VMEM((CB, D), jnp.float32),
                   pltpu.VMEM((CB, D), jnp.float32),
                   pltpu.VMEM((RPS,), jnp.float32),
                   pltpu.VMEM((CB,), jnp.float32),
                   pltpu.VMEM((16,), jnp.int32),
                   pltpu.VMEM((DP,), jnp.int32),
                   pltpu.VMEM_SHARED((NP, D), jnp.float32),
                   pltpu.VMEM_SHARED((NP,), jnp.float32),
                   pltpu.SemaphoreType.DMA,
                   pltpu.SemaphoreType.DMA]
    else:
        out_type = jax.ShapeDtypeStruct((NC, NP, D), jnp.float32)
        scratch = [pltpu.VMEM((G, CB), jnp.int32),
                   pltpu.VMEM((G, CB), jnp.int32),
                   pltpu.VMEM((CB, D), jnp.float32),
                   pltpu.VMEM((CB, D), jnp.float32),
                   pltpu.VMEM_SHARED((NP, D), jnp.float32),
                   pltpu.SemaphoreType.DMA,
                   pltpu.SemaphoreType.DMA]
    return pl.kernel(functools.partial(_scatter_body, with_extras),
                     out_type=out_type, mesh=_mesh, scratch_types=scratch)


_scatter_extras = _build_scatter(True)
_scatter_plain = _build_scatter(False)


BC = 1024


def _combine_body(p_ref, c0_ref, c1_ref, o_ref):
    r = 1.0 / jnp.maximum(c0_ref[...] + c1_ref[...], 1.0)
    o_ref[...] = (p_ref[0] + p_ref[1]) * r


def _combine(part, c0, c1):
    return pl.pallas_call(
        _combine_body,
        grid=(NP // BC,),
        in_specs=[pl.BlockSpec((NC, BC, D), lambda i: (0, i, 0)),
                  pl.BlockSpec((BC, 1), lambda i: (i, 0)),
                  pl.BlockSpec((BC, 1), lambda i: (i, 0))],
        out_specs=pl.BlockSpec((BC, D), lambda i: (i, 0)),
        out_shape=jax.ShapeDtypeStruct((NP, D), jnp.float32),
    )(part, c0, c1)


BF = 2000


def _fusion_body(x_ref, a1_ref, p2_ref, c0_ref, c1_ref,
                 w0_ref, b0_ref, w1_ref, b1_ref, w2_ref, b2_ref,
                 wa_ref, ba_ref, wf0_ref, wf1_ref, wf2_ref, bf_ref, o_ref):
    f32 = jnp.float32
    r = 1.0 / jnp.maximum(c0_ref[...] + c1_ref[...], 1.0)
    a2 = (p2_ref[0] + p2_ref[1]) * r
    h0 = jnp.dot(x_ref[...], w0_ref[...], preferred_element_type=f32) + b0_ref[...]
    h1 = jnp.dot(a1_ref[...], w1_ref[...], preferred_element_type=f32) + b1_ref[...]
    h2 = jnp.dot(a2, w2_ref[...], preferred_element_type=f32) + b2_ref[...]
    wa = wa_ref[...]
    logits = (jnp.dot(h0, wa[0:D], preferred_element_type=f32)
              + jnp.dot(h1, wa[D:2 * D], preferred_element_type=f32)
              + jnp.dot(h2, wa[2 * D:3 * D], preferred_element_type=f32)
              + ba_ref[...])
    m = jnp.max(logits, axis=-1, keepdims=True)
    e = jnp.exp(logits - m)
    att = e / jnp.sum(e, axis=-1, keepdims=True)
    o_ref[...] = (jnp.dot(h0 * att[:, 0:1], wf0_ref[...], preferred_element_type=f32)
                  + jnp.dot(h1 * att[:, 1:2], wf1_ref[...], preferred_element_type=f32)
                  + jnp.dot(h2 * att[:, 2:3], wf2_ref[...], preferred_element_type=f32)
                  + bf_ref[...])


def _fusion(x, a1, p2, c0, c1, w0, b0, w1, b1, w2, b2, wa, ba, wf0, wf1, wf2, bf):
    full = lambda shape: pl.BlockSpec(shape, lambda i: tuple(0 for _ in shape))
    return pl.pallas_call(
        _fusion_body,
        grid=(N // BF,),
        in_specs=[pl.BlockSpec((BF, D), lambda i: (i, 0)),
                  pl.BlockSpec((BF, D), lambda i: (i, 0)),
                  pl.BlockSpec((NC, BF, D), lambda i: (0, i, 0)),
                  pl.BlockSpec((BF, 1), lambda i: (i, 0)),
                  pl.BlockSpec((BF, 1), lambda i: (i, 0)),
                  full((D, D)), full((1, D)),
                  full((D, D)), full((1, D)),
                  full((D, D)), full((1, D)),
                  full((3 * D, D)), full((1, D)),
                  full((D, D)), full((D, D)), full((D, D)), full((1, D))],
        out_specs=pl.BlockSpec((BF, D), lambda i: (i, 0)),
        out_shape=jax.ShapeDtypeStruct((N, D), jnp.float32),
    )(x, a1, p2, c0, c1, w0, b0, w1, b1, w2, b2, wa, ba, wf0, wf1, wf2, bf)


def kernel(node_embeddings, edge_index, node_depths, W0, b0, W1, b1, W2, b2,
           Wg1, bg1, Wg2, bg2, Wa, ba, Wf, bf):
    f32 = jnp.float32
    i32 = jnp.int32

    Xp = jnp.concatenate([node_embeddings, jnp.zeros((NP - N, D), f32)], axis=0)
    pad_e = NW * EW - E
    rowp = jnp.concatenate([edge_index[0], jnp.full((pad_e,), N, i32)])
    colp = jnp.concatenate([edge_index[1], jnp.full((pad_e,), NP - 1, i32)])
    row_r = rowp.reshape(NW, CH, CB)
    col_r = colp.reshape(NW, CH, CB)
    dep_r = jnp.concatenate(
        [node_depths, jnp.zeros((NP - N,), i32)]).reshape(NW, DP)

    part1, cnt, dmax = _scatter_extras(Xp, row_r, col_r, dep_r)

    max_depth = jnp.max(dmax).astype(f32)
    sf = jnp.stack([jnp.asarray(N / 5000.0, f32), max_depth / 20.0])[None, :]
    g = jax.nn.relu(sf @ Wg1.T + bg1)
    sw = jax.nn.sigmoid(g @ Wg2.T + bg2)[0]

    c0 = cnt[0][:, None]
    c1 = cnt[1][:, None]
    agg1 = _combine(part1, c0, c1)

    part2 = _scatter_plain(agg1, row_r, col_r)

    w0t = W0.T * sw[0]
    w1t = W1.T * sw[1]
    w2t = W2.T * sw[2]
    b0s = (b0 * sw[0])[None, :]
    b1s = (b1 * sw[1])[None, :]
    b2s = (b2 * sw[2])[None, :]
    waP = jnp.zeros((3 * D, D), f32).at[:, :MAX_HOP].set(Wa.T)
    baP = jnp.full((1, D), -1e30, f32).at[0, :MAX_HOP].set(ba)
    wfT = Wf.T
    return _fusion(node_embeddings, agg1, part2, c0, c1,
                   w0t, b0s, w1t, b1s, w2t, b2s, waP, baP,
                   wfT[0:D], wfT[D:2 * D], wfT[2 * D:3 * D], bf[None, :])


# speedup vs baseline: 11.3318x; 3.8447x over previous
import functools

import jax
import jax.numpy as jnp
from jax import lax
from jax.experimental import pallas as pl
from jax.experimental.pallas import tpu as pltpu
from jax.experimental.pallas import tpu_sc as plsc

N = 10000
D = 128
E = 320000
MAX_HOP = 3

NC = 2
NS = 16
NW = NC * NS

NP = 10240
RPS = NP // NS
EW = 10240
CB = 128
CH = EW // CB
G = 40
DP = NP // NW

_mesh = plsc.VectorSubcoreMesh(
    core_axis_name="c", subcore_axis_name="s", num_cores=NC, num_subcores=NS)


def _scatter_body(with_extras, *refs):
    if with_extras:
        (feat, row_h, col_h, dep_h, part_h, cnt_h, dmax_h,
         row_v, col_v, rows_a, rows_b, z1_v, ones_v, dm_v, dep_v,
         acc_s, cnt_s, gsem_a, gsem_b) = refs
    else:
        (feat, row_h, col_h, part_h,
         row_v, col_v, rows_a, rows_b, acc_s, gsem_a, gsem_b) = refs

    cid = lax.axis_index("c")
    sid = lax.axis_index("s")
    wid = sid * NC + cid
    base = sid * RPS

    zero16f = jnp.zeros((16,), jnp.float32)

    def _zbody(r, carry):
        for j in range(D // 16):
            rows_a[r, pl.ds(j * 16, 16)] = zero16f
        return carry
    lax.fori_loop(0, CB, _zbody, 0)
    for k in range(RPS // CB):
        pltpu.sync_copy(rows_a, acc_s.at[pl.ds(base + k * CB, CB)])

    if with_extras:
        def _z1body(r, carry):
            z1_v[pl.ds(r * 16, 16)] = zero16f
            return carry
        lax.fori_loop(0, RPS // 16, _z1body, 0)
        pltpu.sync_copy(z1_v, cnt_s.at[pl.ds(base, RPS)])

        one16f = jnp.ones((16,), jnp.float32)
        for j in range(CB // 16):
            ones_v[pl.ds(j * 16, 16)] = one16f

        pltpu.sync_copy(dep_h.at[wid], dep_v)
        m = dep_v[pl.ds(0, 16)]
        for t in range(1, DP // 16):
            m = jnp.maximum(m, dep_v[pl.ds(t * 16, 16)])
        dm_v[pl.ds(0, 16)] = m
        pltpu.sync_copy(dm_v, dmax_h.at[wid])

    plsc.subcore_barrier()

    for sci in range(CH // G):
        pltpu.sync_copy(row_h.at[wid, pl.ds(sci * G, G)], row_v)
        pltpu.sync_copy(col_h.at[wid, pl.ds(sci * G, G)], col_v)
        pltpu.async_copy(feat.at[row_v.at[0]], rows_a, gsem_a)
        pltpu.async_copy(feat.at[row_v.at[1]], rows_b, gsem_b)

        def _step(j, rows, gsem):
            pltpu.make_async_copy(feat.at[row_v.at[j]], rows, gsem).wait()
            pltpu.sync_copy(rows, acc_s.at[col_v.at[j]], add=True)
            if with_extras:
                pltpu.sync_copy(ones_v, cnt_s.at[col_v.at[j]], add=True)

            @pl.when(j < G - 2)
            def _():
                pltpu.async_copy(feat.at[row_v.at[j + 2]], rows, gsem)

        def _pair(g, carry):
            _step(2 * g, rows_a, gsem_a)
            _step(2 * g + 1, rows_b, gsem_b)
            return carry
        lax.fori_loop(0, G // 2, _pair, 0)

    plsc.subcore_barrier()
    pltpu.sync_copy(acc_s.at[pl.ds(base, RPS)],
                    part_h.at[cid, pl.ds(base, RPS)])
    if with_extras:
        pltpu.sync_copy(cnt_s.at[pl.ds(base, RPS)],
                        cnt_h.at[cid, pl.ds(base, RPS)])


def _build_scatter(with_extras):
    if with_extras:
        out_type = (jax.ShapeDtypeStruct((NC, NP, D), jnp.float32),
                    jax.ShapeDtypeStruct((NC, NP), jnp.float32),
                    jax.ShapeDtypeStruct((NW, 16), jnp.int32))
        scratch = [pltpu.VMEM((G, CB), jnp.int32),
                   pltpu.VMEM((G, CB), jnp.int32),
                   pltpu.VMEM((CB, D), jnp.float32),
                   pltpu.VMEM((CB, D), jnp.float32),
                   pltpu.VMEM((RPS,), jnp.float32),
                   pltpu.VMEM((CB,), jnp.float32),
                   pltpu.VMEM((16,), jnp.int32),
                   pltpu.VMEM((DP,), jnp.int32),
                   pltpu.VMEM_SHARED((NP, D), jnp.float32),
                   pltpu.VMEM_SHARED((NP,), jnp.float32),
                   pltpu.SemaphoreType.DMA,
                   pltpu.SemaphoreType.DMA]
    else:
        out_type = jax.ShapeDtypeStruct((NC, NP, D), jnp.float32)
        scratch = [pltpu.VMEM((G, CB), jnp.int32),
                   pltpu.VMEM((G, CB), jnp.int32),
                   pltpu.VMEM((CB, D), jnp.float32),
                   pltpu.VMEM((CB, D), jnp.float32),
                   pltpu.VMEM_SHARED((NP, D), jnp.float32),
                   pltpu.SemaphoreType.DMA,
                   pltpu.SemaphoreType.DMA]
    return pl.kernel(functools.partial(_scatter_body, with_extras),
                     out_type=out_type, mesh=_mesh, scratch_types=scratch)


_scatter_extras = _build_scatter(True)
_scatter_plain = _build_scatter(False)


BC = 1024


def _combine_body(p_ref, c0_ref, c1_ref, o_ref):
    r = 1.0 / jnp.maximum(c0_ref[...] + c1_ref[...], 1.0)
    o_ref[...] = (p_ref[0] + p_ref[1]) * r


def _combine(part, c0, c1):
    return pl.pallas_call(
        _combine_body,
        grid=(NP // BC,),
        in_specs=[pl.BlockSpec((NC, BC, D), lambda i: (0, i, 0)),
                  pl.BlockSpec((BC, 1), lambda i: (i, 0)),
                  pl.BlockSpec((BC, 1), lambda i: (i, 0))],
        out_specs=pl.BlockSpec((BC, D), lambda i: (i, 0)),
        out_shape=jax.ShapeDtypeStruct((NP, D), jnp.float32),
    )(part, c0, c1)


BF = 2000


def _fusion_body(x_ref, a1_ref, p2_ref, c0_ref, c1_ref,
                 w0_ref, b0_ref, w1_ref, b1_ref, w2_ref, b2_ref,
                 wa_ref, ba_ref, wf0_ref, wf1_ref, wf2_ref, bf_ref, o_ref):
    f32 = jnp.float32
    r = 1.0 / jnp.maximum(c0_ref[...] + c1_ref[...], 1.0)
    a2 = (p2_ref[0] + p2_ref[1]) * r
    h0 = jnp.dot(x_ref[...], w0_ref[...], preferred_element_type=f32) + b0_ref[...]
    h1 = jnp.dot(a1_ref[...], w1_ref[...], preferred_element_type=f32) + b1_ref[...]
    h2 = jnp.dot(a2, w2_ref[...], preferred_element_type=f32) + b2_ref[...]
    wa = wa_ref[...]
    logits = (jnp.dot(h0, wa[0:D], preferred_element_type=f32)
              + jnp.dot(h1, wa[D:2 * D], preferred_element_type=f32)
              + jnp.dot(h2, wa[2 * D:3 * D], preferred_element_type=f32)
              + ba_ref[...])
    m = jnp.max(logits, axis=-1, keepdims=True)
    e = jnp.exp(logits - m)
    att = e / jnp.sum(e, axis=-1, keepdims=True)
    o_ref[...] = (jnp.dot(h0 * att[:, 0:1], wf0_ref[...], preferred_element_type=f32)
                  + jnp.dot(h1 * att[:, 1:2], wf1_ref[...], preferred_element_type=f32)
                  + jnp.dot(h2 * att[:, 2:3], wf2_ref[...], preferred_element_type=f32)
                  + bf_ref[...])


def _fusion(x, a1, p2, c0, c1, w0, b0, w1, b1, w2, b2, wa, ba, wf0, wf1, wf2, bf):
    full = lambda shape: pl.BlockSpec(shape, lambda i: tuple(0 for _ in shape))
    return pl.pallas_call(
        _fusion_body,
        grid=(N // BF,),
        in_specs=[pl.BlockSpec((BF, D), lambda i: (i, 0)),
                  pl.BlockSpec((BF, D), lambda i: (i, 0)),
                  pl.BlockSpec((NC, BF, D), lambda i: (0, i, 0)),
                  pl.BlockSpec((BF, 1), lambda i: (i, 0)),
                  pl.BlockSpec((BF, 1), lambda i: (i, 0)),
                  full((D, D)), full((1, D)),
                  full((D, D)), full((1, D)),
                  full((D, D)), full((1, D)),
                  full((3 * D, D)), full((1, D)),
                  full((D, D)), full((D, D)), full((D, D)), full((1, D))],
        out_specs=pl.BlockSpec((BF, D), lambda i: (i, 0)),
        out_shape=jax.ShapeDtypeStruct((N, D), jnp.float32),
    )(x, a1, p2, c0, c1, w0, b0, w1, b1, w2, b2, wa, ba, wf0, wf1, wf2, bf)


def kernel(node_embeddings, edge_index, node_depths, W0, b0, W1, b1, W2, b2,
           Wg1, bg1, Wg2, bg2, Wa, ba, Wf, bf):
    f32 = jnp.float32
    i32 = jnp.int32

    Xp = jnp.concatenate([node_embeddings, jnp.zeros((NP - N, D), f32)], axis=0)
    pad_e = NW * EW - E
    pad_idx = N + (jnp.arange(pad_e, dtype=i32) % (NP - N))
    rowp = jnp.concatenate([edge_index[0], pad_idx])
    colp = jnp.concatenate([edge_index[1], pad_idx])
    row_r = rowp.reshape(NW, CH, CB)
    col_r = colp.reshape(NW, CH, CB)
    dep_r = jnp.concatenate(
        [node_depths, jnp.zeros((NP - N,), i32)]).reshape(NW, DP)

    part1, cnt, dmax = _scatter_extras(Xp, row_r, col_r, dep_r)

    max_depth = jnp.max(dmax).astype(f32)
    sf = jnp.stack([jnp.asarray(N / 5000.0, f32), max_depth / 20.0])[None, :]
    g = jax.nn.relu(sf @ Wg1.T + bg1)
    sw = jax.nn.sigmoid(g @ Wg2.T + bg2)[0]

    c0 = cnt[0][:, None]
    c1 = cnt[1][:, None]
    agg1 = _combine(part1, c0, c1)

    part2 = _scatter_plain(agg1, row_r, col_r)

    w0t = W0.T * sw[0]
    w1t = W1.T * sw[1]
    w2t = W2.T * sw[2]
    b0s = (b0 * sw[0])[None, :]
    b1s = (b1 * sw[1])[None, :]
    b2s = (b2 * sw[2])[None, :]
    waP = jnp.zeros((3 * D, D), f32).at[:, :MAX_HOP].set(Wa.T)
    baP = jnp.full((1, D), -1e30, f32).at[0, :MAX_HOP].set(ba)
    wfT = Wf.T
    return _fusion(node_embeddings, agg1, part2, c0, c1,
                   w0t, b0s, w1t, b1s, w2t, b2s, waP, baP,
                   wfT[0:D], wfT[D:2 * D], wfT[2 * D:3 * D], bf[None, :])
